# baseline (speedup 1.0000x reference)
import functools

import jax
import jax.numpy as jnp
from jax import lax
from jax.experimental import pallas as pl
from jax.experimental.pallas import tpu as pltpu
from jax.experimental.pallas import tpu_sc as plsc

D = 64
S = 200
BT = 4096
NW = 32
BB = BT // NW


def _make_gather():
  mesh = plsc.VectorSubcoreMesh(core_axis_name="c", subcore_axis_name="s")

  @functools.partial(
      pl.kernel,
      mesh=mesh,
      out_type=jax.ShapeDtypeStruct((S, D, BT), jnp.float32),
      compiler_params=pltpu.CompilerParams(
          use_tc_tiling_on_sc=False, needs_layout_passes=False),
      scratch_types=[
          pltpu.VMEM((S, BB), jnp.int32),
          pltpu.VMEM((BB, D), jnp.float32),
          pltpu.VMEM((BB, D), jnp.float32),
          pltpu.VMEM((D, BB), jnp.float32),
          pltpu.VMEM((D, BB), jnp.float32),
          pltpu.SemaphoreType.DMA,
          pltpu.SemaphoreType.DMA,
          pltpu.SemaphoreType.DMA,
          pltpu.SemaphoreType.DMA,
      ],
  )
  def gather_kernel(table_hbm, idx_hbm, out_hbm,
                    idx_v, rows0, rows1, tr0, tr1,
                    gsem0, gsem1, osem0, osem1):
    w = lax.axis_index("s") * 2 + lax.axis_index("c")
    b0 = w * BB
    rows = (rows0, rows1)
    trs = (tr0, tr1)
    gsems = (gsem0, gsem1)
    osems = (osem0, osem1)

    pltpu.sync_copy(idx_hbm.at[:, pl.ds(b0, BB)], idx_v)

    lane = lax.iota(jnp.int32, 16)

    def transpose_block(src, dst):
      def dbody(d, carry):
        col = jnp.full((16,), 0, jnp.int32) + d
        for g in range(BB // 16):
          v = plsc.load_gather(src, [g * 16 + lane, col])
          dst[d, pl.ds(g * 16, 16)] = v
        return carry
      lax.fori_loop(0, D, dbody, 0)

    def start_gather(s, u):
      pltpu.async_copy(table_hbm.at[idx_v.at[s]], rows[u], gsems[u])

    def wait_gather(s, u):
      pltpu.make_async_copy(table_hbm.at[idx_v.at[s]], rows[u],
                            gsems[u]).wait()

    def start_out(s, u):
      pltpu.async_copy(trs[u], out_hbm.at[s, :, pl.ds(b0, BB)], osems[u])

    def wait_out(s, u):
      pltpu.make_async_copy(trs[u], out_hbm.at[s, :, pl.ds(b0, BB)],
                            osems[u]).wait()

    start_gather(0, 0)
    start_gather(1, 1)
    wait_gather(0, 0)
    transpose_block(rows0, tr0)
    start_gather(2, 0)
    start_out(0, 0)
    wait_gather(1, 1)
    transpose_block(rows1, tr1)
    start_gather(3, 1)
    start_out(1, 1)

    def body(sp, carry):
      for u in range(2):
        s = 2 * sp + u
        wait_gather(s, u)
        wait_out(s - 2, u)
        transpose_block(rows[u], trs[u])
        start_out(s, u)

        @pl.when(s + 2 < S)
        def _():
          start_gather(s + 2, u)
      return carry

    lax.fori_loop(1, S // 2, body, 0)

    wait_out(S - 2, 0)
    wait_out(S - 1, 1)

  return gather_kernel


def kernel(x, table):
  idx_sb = x.T.astype(jnp.int32)
  out = _make_gather()(table, idx_sb)
  return out.transpose(2, 0, 1)


# speedup vs baseline: 1.3598x; 1.3598x over previous
import functools

import jax
import jax.numpy as jnp
from jax import lax
from jax.experimental import pallas as pl
from jax.experimental.pallas import tpu as pltpu
from jax.experimental.pallas import tpu_sc as plsc

D = 64
S = 200
BT = 4096
NW = 32
BB = BT // NW


def _make_gather():
  mesh = plsc.VectorSubcoreMesh(core_axis_name="c", subcore_axis_name="s")

  @functools.partial(
      pl.kernel,
      mesh=mesh,
      out_type=jax.ShapeDtypeStruct((S, D, BT), jnp.float32),
      compiler_params=pltpu.CompilerParams(
          use_tc_tiling_on_sc=False, needs_layout_passes=False),
      scratch_types=[
          pltpu.VMEM((S, BB), jnp.int32),
          pltpu.VMEM((BB, D), jnp.float32),
          pltpu.VMEM((BB, D), jnp.float32),
          pltpu.VMEM((D, BB), jnp.float32),
          pltpu.VMEM((D, BB), jnp.float32),
          pltpu.SemaphoreType.DMA,
          pltpu.SemaphoreType.DMA,
          pltpu.SemaphoreType.DMA,
          pltpu.SemaphoreType.DMA,
      ],
  )
  def gather_kernel(table_hbm, idx_hbm, out_hbm,
                    idx_v, rows0, rows1, tr0, tr1,
                    gsem0, gsem1, osem0, osem1):
    w = lax.axis_index("s") * 2 + lax.axis_index("c")
    b0 = w * BB
    rows = (rows0, rows1)
    trs = (tr0, tr1)
    gsems = (gsem0, gsem1)
    osems = (osem0, osem1)

    pltpu.sync_copy(idx_hbm.at[:, pl.ds(b0, BB)], idx_v)

    lane = lax.iota(jnp.int32, 16)

    def transpose_block(src, dst):
      @plsc.parallel_loop(0, D, 1, unroll=8)
      def dbody(d):
        col = jnp.full((16,), 0, jnp.int32) + d
        for g in range(BB // 16):
          v = plsc.load_gather(src, [g * 16 + lane, col])
          dst[d, pl.ds(g * 16, 16)] = v

    def start_gather(s, u):
      pltpu.async_copy(table_hbm.at[idx_v.at[s]], rows[u], gsems[u])

    def wait_gather(s, u):
      pltpu.make_async_copy(table_hbm.at[idx_v.at[s]], rows[u],
                            gsems[u]).wait()

    def start_out(s, u):
      pltpu.async_copy(trs[u], out_hbm.at[s, :, pl.ds(b0, BB)], osems[u])

    def wait_out(s, u):
      pltpu.make_async_copy(trs[u], out_hbm.at[s, :, pl.ds(b0, BB)],
                            osems[u]).wait()

    start_gather(0, 0)
    start_gather(1, 1)
    wait_gather(0, 0)
    transpose_block(rows0, tr0)
    start_gather(2, 0)
    start_out(0, 0)
    wait_gather(1, 1)
    transpose_block(rows1, tr1)
    start_gather(3, 1)
    start_out(1, 1)

    def body(sp, carry):
      for u in range(2):
        s = 2 * sp + u
        wait_gather(s, u)
        wait_out(s - 2, u)
        transpose_block(rows[u], trs[u])
        start_out(s, u)

        @pl.when(s + 2 < S)
        def _():
          start_gather(s + 2, u)
      return carry

    lax.fori_loop(1, S // 2, body, 0)

    wait_out(S - 2, 0)
    wait_out(S - 1, 1)

  return gather_kernel


def kernel(x, table):
  idx_sb = x.T.astype(jnp.int32)
  out = _make_gather()(table, idx_sb)
  return out.transpose(2, 0, 1)


# speedup vs baseline: 2.0516x; 1.5088x over previous
import functools

import jax
import jax.numpy as jnp
from jax import lax
from jax.experimental import pallas as pl
from jax.experimental.pallas import tpu as pltpu
from jax.experimental.pallas import tpu_sc as plsc

D = 64
S = 200
BT = 4096
NW = 32
BB = BT // NW


def _make_gather():
  mesh = plsc.VectorSubcoreMesh(core_axis_name="c", subcore_axis_name="s")

  @functools.partial(
      pl.kernel,
      mesh=mesh,
      out_type=jax.ShapeDtypeStruct((S, D, BT), jnp.float32),
      compiler_params=pltpu.CompilerParams(
          use_tc_tiling_on_sc=False, needs_layout_passes=False),
      scratch_types=[
          pltpu.VMEM((S, BB), jnp.int32),
          pltpu.VMEM((BB, D), jnp.float32),
          pltpu.VMEM((BB, D), jnp.float32),
          pltpu.VMEM((D, BB), jnp.float32),
          pltpu.VMEM((D, BB), jnp.float32),
          pltpu.SemaphoreType.DMA,
          pltpu.SemaphoreType.DMA,
          pltpu.SemaphoreType.DMA,
          pltpu.SemaphoreType.DMA,
      ],
  )
  def gather_kernel(table_hbm, idx_hbm, out_hbm,
                    idx_v, rows0, rows1, tr0, tr1,
                    gsem0, gsem1, osem0, osem1):
    w = lax.axis_index("s") * 2 + lax.axis_index("c")
    b0 = w * BB
    rows = (rows0, rows1)
    trs = (tr0, tr1)
    gsems = (gsem0, gsem1)
    osems = (osem0, osem1)

    pltpu.sync_copy(idx_hbm.at[:, pl.ds(b0, BB)], idx_v)

    lane = lax.iota(jnp.int32, 16)

    def transpose_block(src, dst):
      @plsc.parallel_loop(0, D, 1, unroll=8)
      def dbody(d0):
        colv = (d0 + lane) & (D - 1)
        for g in range(BB // 16):
          rowv = g * 16 + lane
          v = plsc.load_gather(src, [rowv, colv])
          plsc.store_scatter(dst, [colv, rowv], v)

    def start_gather(s, u):
      pltpu.async_copy(table_hbm.at[idx_v.at[s]], rows[u], gsems[u])

    def wait_gather(s, u):
      pltpu.make_async_copy(table_hbm.at[idx_v.at[s]], rows[u],
                            gsems[u]).wait()

    def start_out(s, u):
      pltpu.async_copy(trs[u], out_hbm.at[s, :, pl.ds(b0, BB)], osems[u])

    def wait_out(s, u):
      pltpu.make_async_copy(trs[u], out_hbm.at[s, :, pl.ds(b0, BB)],
                            osems[u]).wait()

    start_gather(0, 0)
    start_gather(1, 1)
    wait_gather(0, 0)
    transpose_block(rows0, tr0)
    start_gather(2, 0)
    start_out(0, 0)
    wait_gather(1, 1)
    transpose_block(rows1, tr1)
    start_gather(3, 1)
    start_out(1, 1)

    def body(sp, carry):
      for u in range(2):
        s = 2 * sp + u
        wait_gather(s, u)
        wait_out(s - 2, u)
        transpose_block(rows[u], trs[u])
        start_out(s, u)

        @pl.when(s + 2 < S)
        def _():
          start_gather(s + 2, u)
      return carry

    lax.fori_loop(1, S // 2, body, 0)

    wait_out(S - 2, 0)
    wait_out(S - 1, 1)

  return gather_kernel


def kernel(x, table):
  idx_sb = x.T.astype(jnp.int32)
  out = _make_gather()(table, idx_sb)
  return out.transpose(2, 0, 1)


# speedup vs baseline: 2.3875x; 1.1637x over previous
import functools

import jax
import jax.numpy as jnp
from jax import lax
from jax.experimental import pallas as pl
from jax.experimental.pallas import tpu as pltpu
from jax.experimental.pallas import tpu_sc as plsc

D = 64
S = 200
BT = 4096
NW = 32
BB = BT // NW
VROWS = 500000


def _make_gather():
  mesh = plsc.VectorSubcoreMesh(core_axis_name="c", subcore_axis_name="s")

  @functools.partial(
      pl.kernel,
      mesh=mesh,
      out_type=jax.ShapeDtypeStruct((S, D // 8, NW, 8, BB), jnp.float32),
      compiler_params=pltpu.CompilerParams(
          use_tc_tiling_on_sc=True, needs_layout_passes=False),
      scratch_types=[
          pltpu.VMEM((S, BB), jnp.int32),
          pltpu.VMEM((S, BB), jnp.int32),
          pltpu.VMEM((BB, 2 * D), jnp.float32),
          pltpu.VMEM((BB, 2 * D), jnp.float32),
          pltpu.VMEM((D // 8, 8, BB), jnp.float32),
          pltpu.VMEM((D // 8, 8, BB), jnp.float32),
          pltpu.SemaphoreType.DMA,
          pltpu.SemaphoreType.DMA,
          pltpu.SemaphoreType.DMA,
          pltpu.SemaphoreType.DMA,
      ],
  )
  def gather_kernel(table_hbm, idx_hbm, out_hbm,
                    idx_v, idx2_v, rows0, rows1, tr0, tr1,
                    gsem0, gsem1, osem0, osem1):
    w = lax.axis_index("s") * 2 + lax.axis_index("c")
    b0 = w * BB
    rows = (rows0, rows1)
    trs = (tr0, tr1)
    gsems = (gsem0, gsem1)
    osems = (osem0, osem1)

    pltpu.sync_copy(idx_hbm.at[:, pl.ds(b0, BB)], idx_v)

    lane = lax.iota(jnp.int32, 16)

    @plsc.parallel_loop(0, S, 1, unroll=4)
    def _(s):
      for g in range(BB // 16):
        v = idx_v[s, pl.ds(g * 16, 16)]
        idx2_v[s, pl.ds(g * 16, 16)] = lax.shift_right_logical(v, 1)

    def transpose_block(src, dst, s):
      par = []
      for g in range(BB // 16):
        par.append((idx_v[s, pl.ds(g * 16, 16)] & 1) << 6)

      @plsc.parallel_loop(0, D, 1, unroll=8)
      def dbody(d0):
        colv = (d0 + lane) & (D - 1)
        dtv = lax.shift_right_logical(colv, 3)
        d8v = colv & 7
        for g in range(BB // 16):
          rowv = g * 16 + lane
          v = plsc.load_gather(src, [rowv, colv + par[g]])
          plsc.store_scatter(dst, [dtv, d8v, rowv], v)

    def start_gather(s, u):
      pltpu.async_copy(table_hbm.at[idx2_v.at[s]], rows[u], gsems[u])

    def wait_gather(s, u):
      pltpu.make_async_copy(table_hbm.at[idx2_v.at[s]], rows[u],
                            gsems[u]).wait()

    def start_out(s, u):
      pltpu.async_copy(trs[u], out_hbm.at[s, :, w, :, :], osems[u])

    def wait_out(s, u):
      pltpu.make_async_copy(trs[u], out_hbm.at[s, :, w, :, :],
                            osems[u]).wait()

    start_gather(0, 0)
    start_gather(1, 1)
    wait_gather(0, 0)
    transpose_block(rows0, tr0, 0)
    start_gather(2, 0)
    start_out(0, 0)
    wait_gather(1, 1)
    transpose_block(rows1, tr1, 1)
    start_gather(3, 1)
    start_out(1, 1)

    def body(sp, carry):
      for u in range(2):
        s = 2 * sp + u
        wait_gather(s, u)
        wait_out(s - 2, u)
        transpose_block(rows[u], trs[u], s)
        start_out(s, u)

        @pl.when(s + 2 < S)
        def _():
          start_gather(s + 2, u)
      return carry

    lax.fori_loop(1, S // 2, body, 0)

    wait_out(S - 2, 0)
    wait_out(S - 1, 1)

  return gather_kernel


def kernel(x, table):
  idx_sb = x.T.astype(jnp.int32)
  table2 = table.reshape(VROWS, 2 * D)
  out5 = _make_gather()(table2, idx_sb)
  return out5.transpose(2, 4, 0, 1, 3).reshape(BT, S, D)


# speedup vs baseline: 2.4599x; 1.0303x over previous
import functools

import jax
import jax.numpy as jnp
from jax import lax
from jax.experimental import pallas as pl
from jax.experimental.pallas import tpu as pltpu
from jax.experimental.pallas import tpu_sc as plsc

D = 64
S = 200
BT = 4096
NW = 32
BB = BT // NW
VROWS = 500000
NG = 4


def _make_gather():
  mesh = plsc.VectorSubcoreMesh(core_axis_name="c", subcore_axis_name="s")

  @functools.partial(
      pl.kernel,
      mesh=mesh,
      out_type=jax.ShapeDtypeStruct((S, D // 8, NW, 8, BB), jnp.float32),
      compiler_params=pltpu.CompilerParams(
          use_tc_tiling_on_sc=True, needs_layout_passes=False),
      scratch_types=[
          pltpu.VMEM((S, BB), jnp.int32),
          pltpu.VMEM((NG, BB), jnp.int32),
          pltpu.VMEM((BB, 2 * D), jnp.float32),
          pltpu.VMEM((BB, 2 * D), jnp.float32),
          pltpu.VMEM((BB, 2 * D), jnp.float32),
          pltpu.VMEM((BB, 2 * D), jnp.float32),
          pltpu.VMEM((D // 8, 8, BB), jnp.float32),
          pltpu.VMEM((D // 8, 8, BB), jnp.float32),
          pltpu.SemaphoreType.DMA,
          pltpu.SemaphoreType.DMA,
          pltpu.SemaphoreType.DMA,
          pltpu.SemaphoreType.DMA,
          pltpu.SemaphoreType.DMA,
          pltpu.SemaphoreType.DMA,
      ],
  )
  def gather_kernel(table_hbm, idx_hbm, out_hbm,
                    idx_v, idx2_v, rows0, rows1, rows2, rows3, tr0, tr1,
                    gsem0, gsem1, gsem2, gsem3, osem0, osem1):
    w = lax.axis_index("s") * 2 + lax.axis_index("c")
    b0 = w * BB
    rows = (rows0, rows1, rows2, rows3)
    trs = (tr0, tr1)
    gsems = (gsem0, gsem1, gsem2, gsem3)
    osems = (osem0, osem1)

    pltpu.sync_copy(idx_hbm.at[:, pl.ds(b0, BB)], idx_v)

    lane = lax.iota(jnp.int32, 16)

    def compute_idx2(s, ug):
      for g in range(BB // 16):
        v = idx_v[s, pl.ds(g * 16, 16)]
        idx2_v[ug, pl.ds(g * 16, 16)] = lax.shift_right_logical(v, 1)

    def transpose_block(src, dst, s):
      par = []
      for g in range(BB // 16):
        par.append((idx_v[s, pl.ds(g * 16, 16)] & 1) << 6)

      @plsc.parallel_loop(0, D, 1, unroll=8)
      def dbody(d0):
        colv = (d0 + lane) & (D - 1)
        dtv = lax.shift_right_logical(colv, 3)
        d8v = colv & 7
        for g in range(BB // 16):
          rowv = g * 16 + lane
          v = plsc.load_gather(src, [rowv, colv + par[g]])
          plsc.store_scatter(dst, [dtv, d8v, rowv], v)

    def start_gather(s, ug):
      pltpu.async_copy(table_hbm.at[idx2_v.at[ug]], rows[ug], gsems[ug])

    def wait_gather(s, ug):
      pltpu.make_async_copy(table_hbm.at[idx2_v.at[ug]], rows[ug],
                            gsems[ug]).wait()

    def start_out(s, u):
      pltpu.async_copy(trs[u], out_hbm.at[s, :, w, :, :], osems[u])

    def wait_out(s, u):
      pltpu.make_async_copy(trs[u], out_hbm.at[s, :, w, :, :],
                            osems[u]).wait()

    def feed(s, ug):
      compute_idx2(s, ug)
      start_gather(s, ug)

    def step(s, ug, uo, first, last):
      wait_gather(s, ug)
      if not first:
        wait_out(s - 2, uo)
      transpose_block(rows[ug], trs[uo], s)
      start_out(s, uo)
      if last:
        @pl.when(s + 3 < S)
        def _():
          feed(s + 3, (ug + 3) % NG)
      else:
        feed(s + 3, (ug + 3) % NG)

    for s in range(3):
      feed(s, s)
    step(0, 0, 0, True, False)
    step(1, 1, 1, True, False)
    step(2, 2, 0, False, False)
    step(3, 3, 1, False, False)

    def body(q, carry):
      for r in range(4):
        s = 4 * q + r
        step(s, r, r % 2, False, True)
      return carry

    lax.fori_loop(1, S // 4, body, 0)

    wait_out(S - 2, 0)
    wait_out(S - 1, 1)

  return gather_kernel


def kernel(x, table):
  idx_sb = x.T.astype(jnp.int32)
  table2 = table.reshape(VROWS, 2 * D)
  out5 = _make_gather()(table2, idx_sb)
  return out5.transpose(2, 4, 0, 1, 3).reshape(BT, S, D)
